# baseline (speedup 1.0000x reference)
import functools

import jax
import jax.numpy as jnp
from jax import lax
from jax.experimental import pallas as pl
from jax.experimental.pallas import tpu as pltpu
from jax.experimental.pallas import tpu_sc as plsc

N = 10000
E = 320000
D = 128
HH = 128
HD = HH // 2

NC = 2
NS = 16
NW = NC * NS
LN = 16

NP = 10240
CH = 128
EPS = 20480
NCH_S = EPS // CH
EPT = 10240
NCH_W = EPT // CH
EP = NW * EPT
RPT = NP // NS

BS = 1024
GRID = NP // BS

_mesh = plsc.VectorSubcoreMesh(core_axis_name="c", subcore_axis_name="s")


def _reduce16(acc_v, part_sh, tmp_v, red_v, sid):
    pltpu.sync_copy(acc_v, part_sh.at[sid])
    plsc.subcore_barrier()
    base = sid * RPT
    pltpu.sync_copy(part_sh.at[0, pl.ds(base, RPT)], red_v)

    def kbody(k, carry):
        pltpu.sync_copy(part_sh.at[k, pl.ds(base, RPT)], tmp_v)

        def abody(i, c2):
            red_v[pl.ds(i * LN, LN)] = (
                red_v[pl.ds(i * LN, LN)] + tmp_v[pl.ds(i * LN, LN)])
            return c2

        lax.fori_loop(0, RPT // LN, abody, 0)
        return carry

    lax.fori_loop(1, NS, kbody, 0)


@functools.partial(
    pl.kernel,
    out_type=jax.ShapeDtypeStruct((NC, NP), jnp.float32),
    mesh=_mesh,
    compiler_params=pltpu.CompilerParams(needs_layout_passes=False),
    scratch_types=[
        pltpu.VMEM((NCH_W, CH), jnp.int32),
        pltpu.VMEM((NP,), jnp.float32),
        pltpu.VMEM((RPT,), jnp.float32),
        pltpu.VMEM((RPT,), jnp.float32),
        pltpu.VMEM_SHARED((NS, NP), jnp.float32),
    ],
)
def _deg_kernel(col_hbm, out_hbm, col_v, hist_v, tmp_v, red_v, part_sh):
    cid = lax.axis_index("c")
    sid = lax.axis_index("s")
    wid = cid * NS + sid
    pltpu.sync_copy(col_hbm.at[wid], col_v)

    def zero_body(i, carry):
        hist_v[pl.ds(i * LN, LN)] = jnp.zeros((LN,), jnp.float32)
        return carry

    lax.fori_loop(0, NP // LN, zero_body, 0)

    ones = jnp.ones((LN,), jnp.float32)

    def body(j, carry):
        for k in range(CH // LN):
            plsc.addupdate_scatter(hist_v, [col_v[j, pl.ds(k * LN, LN)]], ones)
        return carry

    lax.fori_loop(0, NCH_W, body, 0)
    _reduce16(hist_v, part_sh, tmp_v, red_v, sid)
    pltpu.sync_copy(red_v, out_hbm.at[cid, pl.ds(sid * RPT, RPT)])


@functools.partial(
    pl.kernel,
    out_type=jax.ShapeDtypeStruct((NC, NP, HD), jnp.float32),
    mesh=_mesh,
    compiler_params=pltpu.CompilerParams(
        needs_layout_passes=False, use_tc_tiling_on_sc=False),
    scratch_types=[
        pltpu.VMEM((NCH_S, CH), jnp.int32),
        pltpu.VMEM((NCH_S, CH), jnp.int32),
        pltpu.VMEM((4, CH, HD), jnp.float32),
        pltpu.VMEM_SHARED((NP, HD), jnp.float32),
        pltpu.SemaphoreType.DMA,
        pltpu.SemaphoreType.DMA,
    ],
)
def _msg_kernel(g_hbm, row_hbm, col_hbm, out_hbm, row_v, col_v, rows_v, acc_sh,
                gsem, ssem):
    cid = lax.axis_index("c")
    sid = lax.axis_index("s")
    gc = g_hbm.at[cid]
    pltpu.sync_copy(row_hbm.at[sid], row_v)
    pltpu.sync_copy(col_hbm.at[sid], col_v)

    def zero_body(i, carry):
        for k in range(HD // LN):
            rows_v[0, i, pl.ds(k * LN, LN)] = jnp.zeros((LN,), jnp.float32)
        return carry

    lax.fori_loop(0, CH, zero_body, 0)

    def init_body(r, carry):
        pltpu.sync_copy(rows_v.at[0], acc_sh.at[pl.ds(sid * RPT + r * CH, CH)])
        return carry

    lax.fori_loop(0, RPT // CH, init_body, 0)
    plsc.subcore_barrier()

    def body(j, carry):
        @pl.when(j < NCH_S)
        def _():
            @pl.when(j >= 4)
            def _():
                jd = j - 4
                pltpu.make_async_copy(
                    rows_v.at[jd % 4], acc_sh.at[col_v.at[jd]], ssem).wait()

            pltpu.async_copy(gc.at[row_v.at[j]], rows_v.at[j % 4], gsem)

        @pl.when(j > 0)
        def _():
            jp = j - 1
            pltpu.make_async_copy(
                gc.at[row_v.at[jp]], rows_v.at[jp % 4], gsem).wait()
            pltpu.async_copy(rows_v.at[jp % 4], acc_sh.at[col_v.at[jp]], ssem,
                             add=True)

        return carry

    lax.fori_loop(0, NCH_S + 1, body, 0)

    def drain_body(t, carry):
        jd = NCH_S - 4 + t
        pltpu.make_async_copy(
            rows_v.at[jd % 4], acc_sh.at[col_v.at[jd]], ssem).wait()
        return carry

    lax.fori_loop(0, 4, drain_body, 0)
    plsc.subcore_barrier()

    def wb_body(r, carry):
        s = sid * RPT + r * CH
        pltpu.sync_copy(acc_sh.at[pl.ds(s, CH)], out_hbm.at[cid, pl.ds(s, CH)])
        return carry

    lax.fori_loop(0, RPT // CH, wb_body, 0)


@functools.partial(
    pl.kernel,
    out_type=jax.ShapeDtypeStruct((NC, NP), jnp.float32),
    mesh=_mesh,
    compiler_params=pltpu.CompilerParams(needs_layout_passes=False),
    scratch_types=[
        pltpu.VMEM((NCH_W, CH), jnp.int32),
        pltpu.VMEM((NCH_W, CH), jnp.int32),
        pltpu.VMEM((NP,), jnp.float32),
        pltpu.VMEM((NP,), jnp.float32),
        pltpu.VMEM((RPT,), jnp.float32),
        pltpu.VMEM((RPT,), jnp.float32),
        pltpu.VMEM((LN,), jnp.float32),
        pltpu.VMEM_SHARED((NS, NP), jnp.float32),
    ],
)
def _scalar_kernel(g2_hbm, dis_hbm, b2_hbm, row_hbm, col_hbm, out_hbm,
                   row_v, col_v, g2_v, acc_v, tmp_v, red_v, b2_v, part_sh):
    cid = lax.axis_index("c")
    sid = lax.axis_index("s")
    wid = cid * NS + sid
    pltpu.sync_copy(row_hbm.at[wid], row_v)
    pltpu.sync_copy(col_hbm.at[wid], col_v)
    pltpu.sync_copy(g2_hbm, g2_v)
    pltpu.sync_copy(b2_hbm, b2_v)

    def zero_body(i, carry):
        acc_v[pl.ds(i * LN, LN)] = jnp.zeros((LN,), jnp.float32)
        return carry

    lax.fori_loop(0, NP // LN, zero_body, 0)

    def body(j, carry):
        for k in range(CH // LN):
            vals = plsc.load_gather(g2_v, [row_v[j, pl.ds(k * LN, LN)]])
            plsc.addupdate_scatter(acc_v, [col_v[j, pl.ds(k * LN, LN)]], vals)
        return carry

    lax.fori_loop(0, NCH_W, body, 0)
    _reduce16(acc_v, part_sh, tmp_v, red_v, sid)

    base = sid * RPT
    pltpu.sync_copy(dis_hbm.at[pl.ds(base, RPT)], tmp_v)
    b2 = b2_v[pl.ds(0, LN)]
    isc1 = jnp.where(cid == 1, 1.0, 0.0).astype(jnp.float32)

    def ebody(i, carry):
        sl = pl.ds(i * LN, LN)
        g2sl = pl.ds(base + i * LN, LN)
        red_v[sl] = (tmp_v[sl] * (red_v[sl] + isc1 * g2_v[g2sl])
                     + isc1 * b2)
        return carry

    lax.fori_loop(0, RPT // LN, ebody, 0)
    pltpu.sync_copy(red_v, out_hbm.at[cid, pl.ds(base, RPT)])


def _lin1_body(x_ref, w1_ref, h0_ref, h1_ref, g2c_ref, dis_ref):
    deg = h0_ref[...] + h1_ref[...] + 1.0
    dis = lax.rsqrt(deg)
    h = lax.dot_general(
        x_ref[...], w1_ref[...], (((1,), (1,)), ((), ())),
        preferred_element_type=jnp.float32,
    )
    g = dis * h
    g2c_ref[0] = g[:, :HD]
    g2c_ref[1] = g[:, HD:]
    dis_ref[...] = dis


_lin1 = pl.pallas_call(
    _lin1_body,
    grid=(GRID,),
    in_specs=[
        pl.BlockSpec((BS, D), lambda i: (i, 0)),
        pl.BlockSpec((HH, D), lambda i: (0, 0)),
        pl.BlockSpec((BS, 1), lambda i: (i, 0)),
        pl.BlockSpec((BS, 1), lambda i: (i, 0)),
    ],
    out_specs=[
        pl.BlockSpec((NC, BS, HD), lambda i: (0, i, 0)),
        pl.BlockSpec((BS, 1), lambda i: (i, 0)),
    ],
    out_shape=[
        jax.ShapeDtypeStruct((NC, NP, HD), jnp.float32),
        jax.ShapeDtypeStruct((NP, 1), jnp.float32),
    ],
)


def _lin2_body(accl_ref, accr_ref, gl_ref, gr_ref, dis_ref, b1_ref, w2_ref,
               g2_ref):
    dis = dis_ref[...]
    outl = dis * (accl_ref[...] + gl_ref[...]) + b1_ref[:, :HD]
    outr = dis * (accr_ref[...] + gr_ref[...]) + b1_ref[:, HD:]
    outl = jnp.maximum(outl, 0.0)
    outr = jnp.maximum(outr, 0.0)
    z = (jnp.sum(outl * w2_ref[:, :HD], axis=1, keepdims=True)
         + jnp.sum(outr * w2_ref[:, HD:], axis=1, keepdims=True))
    g2_ref[...] = dis * z


_lin2 = pl.pallas_call(
    _lin2_body,
    grid=(GRID,),
    in_specs=[
        pl.BlockSpec((BS, HD), lambda i: (i, 0)),
        pl.BlockSpec((BS, HD), lambda i: (i, 0)),
        pl.BlockSpec((BS, HD), lambda i: (i, 0)),
        pl.BlockSpec((BS, HD), lambda i: (i, 0)),
        pl.BlockSpec((BS, 1), lambda i: (i, 0)),
        pl.BlockSpec((1, HH), lambda i: (0, 0)),
        pl.BlockSpec((1, HH), lambda i: (0, 0)),
    ],
    out_specs=pl.BlockSpec((BS, 1), lambda i: (i, 0)),
    out_shape=jax.ShapeDtypeStruct((NP, 1), jnp.float32),
)


def _final_body(e0_ref, e1_ref, out_ref):
    out_ref[...] = e0_ref[...] + e1_ref[...]


_final = pl.pallas_call(
    _final_body,
    grid=(GRID,),
    in_specs=[
        pl.BlockSpec((BS, 1), lambda i: (i, 0)),
        pl.BlockSpec((BS, 1), lambda i: (i, 0)),
    ],
    out_specs=pl.BlockSpec((BS, 1), lambda i: (i, 0)),
    out_shape=jax.ShapeDtypeStruct((NP, 1), jnp.float32),
)


def kernel(x, edge_index, W1, b1, W2, b2):
    row = edge_index[0].astype(jnp.int32)
    col = edge_index[1].astype(jnp.int32)
    pad = EP - E
    pad_iota = jnp.arange(pad, dtype=jnp.int32)
    row_p = jnp.concatenate([row, pad_iota % N])
    col_p = jnp.concatenate([col, N + pad_iota % (NP - N)])
    row3w = row_p.reshape(NW, NCH_W, CH)
    col3w = col_p.reshape(NW, NCH_W, CH)
    row3s = row_p.reshape(NS, NCH_S, CH)
    col3s = col_p.reshape(NS, NCH_S, CH)

    x_p = jnp.pad(x, ((0, NP - N), (0, 0)))

    hist = _deg_kernel(col3w)
    g2c, dis = _lin1(x_p, W1, hist[0].reshape(NP, 1),
                     hist[1].reshape(NP, 1))
    acc = _msg_kernel(g2c, row3s, col3s)
    g2 = _lin2(acc[0], acc[1], g2c[0], g2c[1], dis,
               b1.reshape(1, HH), W2.reshape(1, HH))
    b2b = jnp.broadcast_to(b2, (LN,))
    eparts = _scalar_kernel(g2.reshape(NP), dis.reshape(NP), b2b,
                            row3w, col3w)
    out = _final(eparts[0].reshape(NP, 1), eparts[1].reshape(NP, 1))
    return out.reshape(-1)[:N]


# speedup vs baseline: 1.0606x; 1.0606x over previous
import functools

import jax
import jax.numpy as jnp
from jax import lax
from jax.experimental import pallas as pl
from jax.experimental.pallas import tpu as pltpu
from jax.experimental.pallas import tpu_sc as plsc

N = 10000
E = 320000
D = 128
HH = 128
HD = HH // 2

NC = 2
NS = 16
NW = NC * NS
LN = 16

NP = 10240
CH = 128
EPS = 20480
NCH_S = EPS // CH
EPT = 10240
NCH_W = EPT // CH
EP = NW * EPT
RPT = NP // NS

BS = 1024
GRID = NP // BS

_mesh = plsc.VectorSubcoreMesh(core_axis_name="c", subcore_axis_name="s")


@functools.partial(
    pl.kernel,
    out_type=jax.ShapeDtypeStruct((NW, NP), jnp.float32),
    mesh=_mesh,
    compiler_params=pltpu.CompilerParams(needs_layout_passes=False),
    scratch_types=[
        pltpu.VMEM((NCH_W, CH), jnp.int32),
        pltpu.VMEM((NP,), jnp.float32),
    ],
)
def _deg_kernel(col_hbm, out_hbm, col_v, hist_v):
    cid = lax.axis_index("c")
    sid = lax.axis_index("s")
    wid = cid * NS + sid
    pltpu.sync_copy(col_hbm.at[wid], col_v)

    def zero_body(i, carry):
        hist_v[pl.ds(i * LN, LN)] = jnp.zeros((LN,), jnp.float32)
        return carry

    lax.fori_loop(0, NP // LN, zero_body, 0)

    ones = jnp.ones((LN,), jnp.float32)

    def body(j, carry):
        for k in range(CH // LN):
            plsc.addupdate_scatter(hist_v, [col_v[j, pl.ds(k * LN, LN)]], ones)
        return carry

    lax.fori_loop(0, NCH_W, body, 0)
    pltpu.sync_copy(hist_v, out_hbm.at[wid])


@functools.partial(
    pl.kernel,
    out_type=jax.ShapeDtypeStruct((NC, NP, HD), jnp.float32),
    mesh=_mesh,
    compiler_params=pltpu.CompilerParams(
        needs_layout_passes=False, use_tc_tiling_on_sc=False),
    scratch_types=[
        pltpu.VMEM((NCH_S, CH), jnp.int32),
        pltpu.VMEM((NCH_S, CH), jnp.int32),
        pltpu.VMEM((4, CH, HD), jnp.float32),
        pltpu.VMEM_SHARED((NP, HD), jnp.float32),
        pltpu.SemaphoreType.DMA,
        pltpu.SemaphoreType.DMA,
    ],
)
def _msg_kernel(g_hbm, row_hbm, col_hbm, out_hbm, row_v, col_v, rows_v, acc_sh,
                gsem, ssem):
    cid = lax.axis_index("c")
    sid = lax.axis_index("s")
    gc = g_hbm.at[cid]
    pltpu.sync_copy(row_hbm.at[sid], row_v)
    pltpu.sync_copy(col_hbm.at[sid], col_v)

    def zero_body(i, carry):
        for k in range(HD // LN):
            rows_v[0, i, pl.ds(k * LN, LN)] = jnp.zeros((LN,), jnp.float32)
        return carry

    lax.fori_loop(0, CH, zero_body, 0)

    def init_body(r, carry):
        pltpu.sync_copy(rows_v.at[0], acc_sh.at[pl.ds(sid * RPT + r * CH, CH)])
        return carry

    lax.fori_loop(0, RPT // CH, init_body, 0)
    plsc.subcore_barrier()

    def body(j, carry):
        @pl.when(j < NCH_S)
        def _():
            @pl.when(j >= 4)
            def _():
                jd = j - 4
                pltpu.make_async_copy(
                    rows_v.at[jd % 4], acc_sh.at[col_v.at[jd]], ssem).wait()

            pltpu.async_copy(gc.at[row_v.at[j]], rows_v.at[j % 4], gsem)

        @pl.when(j > 0)
        def _():
            jp = j - 1
            pltpu.make_async_copy(
                gc.at[row_v.at[jp]], rows_v.at[jp % 4], gsem).wait()
            pltpu.async_copy(rows_v.at[jp % 4], acc_sh.at[col_v.at[jp]], ssem,
                             add=True)

        return carry

    lax.fori_loop(0, NCH_S + 1, body, 0)

    def drain_body(t, carry):
        jd = NCH_S - 4 + t
        pltpu.make_async_copy(
            rows_v.at[jd % 4], acc_sh.at[col_v.at[jd]], ssem).wait()
        return carry

    lax.fori_loop(0, 4, drain_body, 0)
    plsc.subcore_barrier()

    def wb_body(r, carry):
        s = sid * RPT + r * CH
        pltpu.sync_copy(acc_sh.at[pl.ds(s, CH)], out_hbm.at[cid, pl.ds(s, CH)])
        return carry

    lax.fori_loop(0, RPT // CH, wb_body, 0)


@functools.partial(
    pl.kernel,
    out_type=jax.ShapeDtypeStruct((NW, NP), jnp.float32),
    mesh=_mesh,
    compiler_params=pltpu.CompilerParams(needs_layout_passes=False),
    scratch_types=[
        pltpu.VMEM((NCH_W, CH), jnp.int32),
        pltpu.VMEM((NCH_W, CH), jnp.int32),
        pltpu.VMEM((NP,), jnp.float32),
        pltpu.VMEM((NP,), jnp.float32),
    ],
)
def _scalar_kernel(g2_hbm, row_hbm, col_hbm, out_hbm, row_v, col_v, g2_v, acc_v):
    cid = lax.axis_index("c")
    sid = lax.axis_index("s")
    wid = cid * NS + sid
    pltpu.sync_copy(row_hbm.at[wid], row_v)
    pltpu.sync_copy(col_hbm.at[wid], col_v)
    pltpu.sync_copy(g2_hbm, g2_v)

    def zero_body(i, carry):
        acc_v[pl.ds(i * LN, LN)] = jnp.zeros((LN,), jnp.float32)
        return carry

    lax.fori_loop(0, NP // LN, zero_body, 0)

    def body(j, carry):
        for k in range(CH // LN):
            vals = plsc.load_gather(g2_v, [row_v[j, pl.ds(k * LN, LN)]])
            plsc.addupdate_scatter(acc_v, [col_v[j, pl.ds(k * LN, LN)]], vals)
        return carry

    lax.fori_loop(0, NCH_W, body, 0)
    pltpu.sync_copy(acc_v, out_hbm.at[wid])


def _lin1_body(x_ref, w1_ref, hist_ref, g2c_ref, dis_ref):
    deg = jnp.sum(hist_ref[...], axis=1, keepdims=True) + 1.0
    dis = lax.rsqrt(deg)
    h = lax.dot_general(
        x_ref[...], w1_ref[...], (((1,), (1,)), ((), ())),
        preferred_element_type=jnp.float32,
    )
    g = dis * h
    g2c_ref[0] = g[:, :HD]
    g2c_ref[1] = g[:, HD:]
    dis_ref[...] = dis


_lin1 = pl.pallas_call(
    _lin1_body,
    grid=(GRID,),
    in_specs=[
        pl.BlockSpec((BS, D), lambda i: (i, 0)),
        pl.BlockSpec((HH, D), lambda i: (0, 0)),
        pl.BlockSpec((BS, NW), lambda i: (i, 0)),
    ],
    out_specs=[
        pl.BlockSpec((NC, BS, HD), lambda i: (0, i, 0)),
        pl.BlockSpec((BS, 1), lambda i: (i, 0)),
    ],
    out_shape=[
        jax.ShapeDtypeStruct((NC, NP, HD), jnp.float32),
        jax.ShapeDtypeStruct((NP, 1), jnp.float32),
    ],
)


def _lin2_body(accl_ref, accr_ref, gl_ref, gr_ref, dis_ref, b1_ref, w2_ref,
               g2_ref):
    dis = dis_ref[...]
    outl = dis * (accl_ref[...] + gl_ref[...]) + b1_ref[:, :HD]
    outr = dis * (accr_ref[...] + gr_ref[...]) + b1_ref[:, HD:]
    outl = jnp.maximum(outl, 0.0)
    outr = jnp.maximum(outr, 0.0)
    z = (jnp.sum(outl * w2_ref[:, :HD], axis=1, keepdims=True)
         + jnp.sum(outr * w2_ref[:, HD:], axis=1, keepdims=True))
    g2_ref[...] = dis * z


_lin2 = pl.pallas_call(
    _lin2_body,
    grid=(GRID,),
    in_specs=[
        pl.BlockSpec((BS, HD), lambda i: (i, 0)),
        pl.BlockSpec((BS, HD), lambda i: (i, 0)),
        pl.BlockSpec((BS, HD), lambda i: (i, 0)),
        pl.BlockSpec((BS, HD), lambda i: (i, 0)),
        pl.BlockSpec((BS, 1), lambda i: (i, 0)),
        pl.BlockSpec((1, HH), lambda i: (0, 0)),
        pl.BlockSpec((1, HH), lambda i: (0, 0)),
    ],
    out_specs=pl.BlockSpec((BS, 1), lambda i: (i, 0)),
    out_shape=jax.ShapeDtypeStruct((NP, 1), jnp.float32),
)


def _final_body(part_ref, g2_ref, dis_ref, b2_ref, out_ref):
    acc = jnp.sum(part_ref[...], axis=1, keepdims=True)
    out_ref[...] = dis_ref[...] * (acc + g2_ref[...]) + b2_ref[...]


_final = pl.pallas_call(
    _final_body,
    grid=(GRID,),
    in_specs=[
        pl.BlockSpec((BS, NW), lambda i: (i, 0)),
        pl.BlockSpec((BS, 1), lambda i: (i, 0)),
        pl.BlockSpec((BS, 1), lambda i: (i, 0)),
        pl.BlockSpec((1, 1), lambda i: (0, 0)),
    ],
    out_specs=pl.BlockSpec((BS, 1), lambda i: (i, 0)),
    out_shape=jax.ShapeDtypeStruct((NP, 1), jnp.float32),
)


def kernel(x, edge_index, W1, b1, W2, b2):
    row = edge_index[0].astype(jnp.int32)
    col = edge_index[1].astype(jnp.int32)
    pad = EP - E
    pad_iota = jnp.arange(pad, dtype=jnp.int32)
    row_p = jnp.concatenate([row, pad_iota % N])
    col_p = jnp.concatenate([col, N + pad_iota % (NP - N)])
    row3w = row_p.reshape(NW, NCH_W, CH)
    col3w = col_p.reshape(NW, NCH_W, CH)
    row3s = row_p.reshape(NS, NCH_S, CH)
    col3s = col_p.reshape(NS, NCH_S, CH)

    x_p = jnp.pad(x, ((0, NP - N), (0, 0)))

    hist = _deg_kernel(col3w)
    g2c, dis = _lin1(x_p, W1, hist.T)
    acc = _msg_kernel(g2c, row3s, col3s)
    g2 = _lin2(acc[0], acc[1], g2c[0], g2c[1], dis,
               b1.reshape(1, HH), W2.reshape(1, HH))
    part2 = _scalar_kernel(g2.reshape(NP), row3w, col3w)
    out = _final(part2.T, g2, dis, b2.reshape(1, 1))
    return out.reshape(-1)[:N]
